# baseline (speedup 1.0000x reference)
import jax
import jax.numpy as jnp
from jax.experimental import pallas as pl
from jax.experimental.pallas import tpu as pltpu

_E = 8
_C = 128
_NY = 64
_NX = 128
_P = 2
_YB = 8


def _fused(x_ref, eta_ref, W_ref, b_ref, Wt_ref, Tmat_ref, out_ref, eta_out_ref):
    Wt_bf = Wt_ref[...].astype(jnp.bfloat16)
    W_bf = W_ref[...]
    xb_all = x_ref[...].astype(jnp.bfloat16)

    for y in range(_YB):
        xb = xb_all[:, y, :]
        eta_y = eta_ref[y, :]

        logits = jax.lax.dot_general(
            xb, Wt_bf, (((0,), (0,)), ((), ())),
            preferred_element_type=jnp.float32)
        tadd = jnp.zeros((_NX, _E), jnp.float32)
        for k in range(_E):
            tadd = jnp.where(eta_y[:, None] == k, Tmat_ref[k][None, :], tadd)
        logits = logits + tadd
        new_eta = jnp.argmax(logits, axis=1).astype(jnp.int32)
        eta_out_ref[y, :] = new_eta

        mask = (new_eta[None, :] == jax.lax.broadcasted_iota(
            jnp.int32, (_E, _NX), 0)).astype(jnp.float32)
        mask_bf = mask.astype(jnp.bfloat16)
        xm = jnp.concatenate([xb * mask_bf[e:e + 1, :] for e in range(_E)],
                             axis=0)

        badd = jax.lax.dot_general(
            b_ref[...], mask, (((1,), (0,)), ((), ())),
            preferred_element_type=jnp.float32)

        for p in range(_P):
            yv = jax.lax.dot_general(
                W_bf[p], xm, (((1,), (0,)), ((), ())),
                preferred_element_type=jnp.float32)
            out_ref[p, :, y, :] = yv + badd[p]


def kernel(x, eta, W, b, Wt, Tmat):
    W_bf = jnp.transpose(W, (0, 2, 1, 3)).reshape(
        _P, _C, _E * _C).astype(jnp.bfloat16)
    grid = (_NY // _YB,)
    out, new_eta = pl.pallas_call(
        _fused,
        grid=grid,
        in_specs=[
            pl.BlockSpec((_C, _YB, _NX), lambda i: (0, i, 0)),
            pl.BlockSpec((_YB, _NX), lambda i: (i, 0)),
            pl.BlockSpec((_P, _C, _E * _C), lambda i: (0, 0, 0)),
            pl.BlockSpec((_P, _E, _C), lambda i: (0, 0, 0)),
            pl.BlockSpec((_C, _E), lambda i: (0, 0)),
            pl.BlockSpec((_E, _E), lambda i: (0, 0)),
        ],
        out_specs=[
            pl.BlockSpec((_P, _C, _YB, _NX), lambda i: (0, 0, i, 0)),
            pl.BlockSpec((_YB, _NX), lambda i: (i, 0)),
        ],
        out_shape=[
            jax.ShapeDtypeStruct((_P, _C, _NY, _NX), jnp.float32),
            jax.ShapeDtypeStruct((_NY, _NX), jnp.int32),
        ],
        compiler_params=pltpu.CompilerParams(
            dimension_semantics=("arbitrary",)),
    )(x, eta, W_bf, b, Wt, Tmat)
    return out, new_eta


# speedup vs baseline: 1.1735x; 1.1735x over previous
import jax
import jax.numpy as jnp
from jax.experimental import pallas as pl
from jax.experimental.pallas import tpu as pltpu

_E = 8
_C = 128
_NY = 64
_NX = 128
_P = 2
_YB = 8


def _fused(x_ref, eta_ref, W_ref, b_ref, Wt_ref, Tmat_ref, out_ref, eta_out_ref):
    Wt_bf = Wt_ref[...]
    W_bf = W_ref[...]
    xb_all = x_ref[...]

    brep = (jax.lax.broadcasted_iota(jnp.int32, (_E * _C, _E), 0) // _C ==
            jax.lax.broadcasted_iota(jnp.int32, (_E * _C, _E), 1)
            ).astype(jnp.bfloat16)

    for y in range(_YB):
        xb = xb_all[:, y, :]
        eta_y = eta_ref[y, :]

        logits = jax.lax.dot_general(
            xb, Wt_bf, (((0,), (0,)), ((), ())),
            preferred_element_type=jnp.float32)
        tadd = jnp.zeros((_NX, _E), jnp.float32)
        for k in range(_E):
            tadd = jnp.where(eta_y[:, None] == k, Tmat_ref[k][None, :], tadd)
        logits = logits + tadd
        new_eta = jnp.argmax(logits, axis=1).astype(jnp.int32)
        eta_out_ref[y, :] = new_eta

        mask = (new_eta[None, :] == jax.lax.broadcasted_iota(
            jnp.int32, (_E, _NX), 0)).astype(jnp.float32)
        mrep = jax.lax.dot_general(
            brep, mask.astype(jnp.bfloat16), (((1,), (0,)), ((), ())),
            preferred_element_type=jnp.float32
            ).astype(jnp.bfloat16)
        xrep = jnp.concatenate([xb] * _E, axis=0)
        xm = xrep * mrep

        badd = jax.lax.dot_general(
            b_ref[...], mask, (((1,), (0,)), ((), ())),
            preferred_element_type=jnp.float32)

        for p in range(_P):
            yv = jax.lax.dot_general(
                W_bf[p], xm, (((1,), (0,)), ((), ())),
                preferred_element_type=jnp.float32)
            out_ref[p, :, y, :] = yv + badd[p]


def kernel(x, eta, W, b, Wt, Tmat):
    xb = x.astype(jnp.bfloat16)
    W_bf = jnp.transpose(W, (0, 2, 1, 3)).reshape(
        _P, _C, _E * _C).astype(jnp.bfloat16)
    grid = (_NY // _YB,)
    out, new_eta = pl.pallas_call(
        _fused,
        grid=grid,
        in_specs=[
            pl.BlockSpec((_C, _YB, _NX), lambda i: (0, i, 0)),
            pl.BlockSpec((_YB, _NX), lambda i: (i, 0)),
            pl.BlockSpec((_P, _C, _E * _C), lambda i: (0, 0, 0)),
            pl.BlockSpec((_P, _E, _C), lambda i: (0, 0, 0)),
            pl.BlockSpec((_C, _E), lambda i: (0, 0)),
            pl.BlockSpec((_E, _E), lambda i: (0, 0)),
        ],
        out_specs=[
            pl.BlockSpec((_P, _C, _YB, _NX), lambda i: (0, 0, i, 0)),
            pl.BlockSpec((_YB, _NX), lambda i: (i, 0)),
        ],
        out_shape=[
            jax.ShapeDtypeStruct((_P, _C, _NY, _NX), jnp.float32),
            jax.ShapeDtypeStruct((_NY, _NX), jnp.int32),
        ],
        compiler_params=pltpu.CompilerParams(
            dimension_semantics=("arbitrary",)),
    )(xb, eta, W_bf, b, Wt.astype(jnp.bfloat16), Tmat)
    return out, new_eta


# speedup vs baseline: 1.6485x; 1.4048x over previous
import jax
import jax.numpy as jnp
from jax.experimental import pallas as pl
from jax.experimental.pallas import tpu as pltpu

_E = 8
_C = 128
_NY = 64
_NX = 128
_P = 2
_YB = 8


def _fused(x_ref, eta_ref, W_ref, b_ref, Wt_ref, Tmat_ref, out_ref, eta_out_ref):
    Wt_bf = Wt_ref[...]
    W_bf = W_ref[...]
    xb_all = x_ref[...]

    brep = (jax.lax.broadcasted_iota(jnp.int32, (_E * _C, _E), 0) // _C ==
            jax.lax.broadcasted_iota(jnp.int32, (_E * _C, _E), 1)
            ).astype(jnp.bfloat16)
    tmat_t = Tmat_ref[...].T
    tcols = [jnp.broadcast_to(tmat_t[:, k:k + 1], (_E, _NX)) for k in range(_E)]
    eidx_sub = jax.lax.broadcasted_iota(jnp.int32, (_E, _NX), 0)

    for y in range(_YB):
        xb = xb_all[:, y, :]
        eta_row = eta_ref[y:y + 1, :]
        etab = jnp.concatenate([eta_row] * _E, axis=0)

        logits = jax.lax.dot_general(
            Wt_bf, xb, (((0,), (0,)), ((), ())),
            preferred_element_type=jnp.float32)
        tadd = jnp.zeros((_E, _NX), jnp.float32)
        for k in range(_E):
            tadd = jnp.where(etab == k, tcols[k], tadd)
        logits = logits + tadd

        mx = jnp.max(logits, axis=0, keepdims=True)
        mxb = jnp.concatenate([mx] * _E, axis=0)
        cand = jnp.where(logits == mxb, eidx_sub, _E)
        new_eta_row = jnp.min(cand, axis=0, keepdims=True)
        eta_out_ref[y:y + 1, :] = new_eta_row

        netab = jnp.concatenate([new_eta_row] * _E, axis=0)
        mask = (netab == eidx_sub).astype(jnp.float32)
        mrep = jax.lax.dot_general(
            brep, mask.astype(jnp.bfloat16), (((1,), (0,)), ((), ())),
            preferred_element_type=jnp.float32
            ).astype(jnp.bfloat16)
        xrep = jnp.concatenate([xb] * _E, axis=0)
        xm = xrep * mrep

        badd = jax.lax.dot_general(
            b_ref[...], mask, (((1,), (0,)), ((), ())),
            preferred_element_type=jnp.float32)

        for p in range(_P):
            yv = jax.lax.dot_general(
                W_bf[p], xm, (((1,), (0,)), ((), ())),
                preferred_element_type=jnp.float32)
            out_ref[p, :, y, :] = yv + badd[p]


def kernel(x, eta, W, b, Wt, Tmat):
    xb = x.astype(jnp.bfloat16)
    W_bf = jnp.transpose(W, (0, 2, 1, 3)).reshape(
        _P, _C, _E * _C).astype(jnp.bfloat16)
    grid = (_NY // _YB,)
    out, new_eta = pl.pallas_call(
        _fused,
        grid=grid,
        in_specs=[
            pl.BlockSpec((_C, _YB, _NX), lambda i: (0, i, 0)),
            pl.BlockSpec((_YB, _NX), lambda i: (i, 0)),
            pl.BlockSpec((_P, _C, _E * _C), lambda i: (0, 0, 0)),
            pl.BlockSpec((_P, _E, _C), lambda i: (0, 0, 0)),
            pl.BlockSpec((_C, _E), lambda i: (0, 0)),
            pl.BlockSpec((_E, _E), lambda i: (0, 0)),
        ],
        out_specs=[
            pl.BlockSpec((_P, _C, _YB, _NX), lambda i: (0, 0, i, 0)),
            pl.BlockSpec((_YB, _NX), lambda i: (i, 0)),
        ],
        out_shape=[
            jax.ShapeDtypeStruct((_P, _C, _NY, _NX), jnp.float32),
            jax.ShapeDtypeStruct((_NY, _NX), jnp.int32),
        ],
        compiler_params=pltpu.CompilerParams(
            dimension_semantics=("arbitrary",)),
    )(xb, eta, W_bf, b, Wt.astype(jnp.bfloat16), Tmat)
    return out, new_eta


# speedup vs baseline: 1.9202x; 1.1649x over previous
import jax
import jax.numpy as jnp
from jax.experimental import pallas as pl
from jax.experimental.pallas import tpu as pltpu

_E = 8
_C = 128
_NY = 64
_NX = 128
_P = 2
_YB = 16


def _fused(x_ref, eta_ref, W_ref, b_ref, Wt_ref, Tmat_ref, out_ref, eta_out_ref):
    Wt_bf = Wt_ref[...]
    xb_all = x_ref[...].astype(jnp.bfloat16)
    Wcat = [jnp.concatenate([W_ref[p, e].astype(jnp.bfloat16)
                             for e in range(_E)], axis=1)
            for p in range(_P)]

    brep = (jax.lax.broadcasted_iota(jnp.int32, (_E * _C, _E), 0) // _C ==
            jax.lax.broadcasted_iota(jnp.int32, (_E * _C, _E), 1)
            ).astype(jnp.bfloat16)
    tmat_t = Tmat_ref[...].T
    tcols = [jnp.broadcast_to(tmat_t[:, k:k + 1], (_E, _NX)) for k in range(_E)]
    eidx_sub = jax.lax.broadcasted_iota(jnp.int32, (_E, _NX), 0)

    for y in range(_YB):
        xb = xb_all[:, y, :]
        eta_row = eta_ref[y:y + 1, :]
        etab = jnp.concatenate([eta_row] * _E, axis=0)

        logits = jax.lax.dot_general(
            Wt_bf, xb, (((0,), (0,)), ((), ())),
            preferred_element_type=jnp.float32)
        tadd = jnp.zeros((_E, _NX), jnp.float32)
        for k in range(_E):
            tadd = jnp.where(etab == k, tcols[k], tadd)
        logits = logits + tadd

        mx = jnp.max(logits, axis=0, keepdims=True)
        mxb = jnp.concatenate([mx] * _E, axis=0)
        cand = jnp.where(logits == mxb, eidx_sub, _E)
        new_eta_row = jnp.min(cand, axis=0, keepdims=True)
        eta_out_ref[y:y + 1, :] = new_eta_row

        netab = jnp.concatenate([new_eta_row] * _E, axis=0)
        mask = (netab == eidx_sub).astype(jnp.float32)
        mrep = jax.lax.dot_general(
            brep, mask.astype(jnp.bfloat16), (((1,), (0,)), ((), ())),
            preferred_element_type=jnp.float32
            ).astype(jnp.bfloat16)
        xrep = jnp.concatenate([xb] * _E, axis=0)
        xm = xrep * mrep

        badd = jax.lax.dot_general(
            b_ref[...], mask, (((1,), (0,)), ((), ())),
            preferred_element_type=jnp.float32)

        for p in range(_P):
            yv = jax.lax.dot_general(
                Wcat[p], xm, (((1,), (0,)), ((), ())),
                preferred_element_type=jnp.float32)
            out_ref[p, :, y, :] = yv + badd[p]


def kernel(x, eta, W, b, Wt, Tmat):
    grid = (_NY // _YB,)
    out, new_eta = pl.pallas_call(
        _fused,
        grid=grid,
        in_specs=[
            pl.BlockSpec((_C, _YB, _NX), lambda i: (0, i, 0)),
            pl.BlockSpec((_YB, _NX), lambda i: (i, 0)),
            pl.BlockSpec((_P, _E, _C, _C), lambda i: (0, 0, 0, 0)),
            pl.BlockSpec((_P, _E, _C), lambda i: (0, 0, 0)),
            pl.BlockSpec((_C, _E), lambda i: (0, 0)),
            pl.BlockSpec((_E, _E), lambda i: (0, 0)),
        ],
        out_specs=[
            pl.BlockSpec((_P, _C, _YB, _NX), lambda i: (0, 0, i, 0)),
            pl.BlockSpec((_YB, _NX), lambda i: (i, 0)),
        ],
        out_shape=[
            jax.ShapeDtypeStruct((_P, _C, _NY, _NX), jnp.float32),
            jax.ShapeDtypeStruct((_NY, _NX), jnp.int32),
        ],
        compiler_params=pltpu.CompilerParams(
            dimension_semantics=("arbitrary",)),
    )(x, eta, W, b, Wt.astype(jnp.bfloat16), Tmat)
    return out, new_eta


# speedup vs baseline: 3.1952x; 1.6640x over previous
import jax
import jax.numpy as jnp
from jax.experimental import pallas as pl
from jax.experimental.pallas import tpu as pltpu

_E = 8
_C = 128
_NY = 64
_NX = 128
_P = 2
_YB = 16
_G = 4
_L = _G * _NX


def _fused(x_ref, eta_ref, W_ref, b_ref, Wt_ref, Tmat_ref, out_ref, eta_out_ref):
    Wt_bf = Wt_ref[...].astype(jnp.bfloat16)
    xb_all = x_ref[...].astype(jnp.bfloat16)
    Wcat = [jnp.concatenate([W_ref[p, e].astype(jnp.bfloat16)
                             for e in range(_E)], axis=1)
            for p in range(_P)]

    brep = (jax.lax.broadcasted_iota(jnp.int32, (_E * _C, _E), 0) // _C ==
            jax.lax.broadcasted_iota(jnp.int32, (_E * _C, _E), 1)
            ).astype(jnp.bfloat16)
    tmat_t = Tmat_ref[...].T
    tcols = [jnp.broadcast_to(tmat_t[:, k:k + 1], (_E, _L)) for k in range(_E)]
    eidx_sub = jax.lax.broadcasted_iota(jnp.int32, (_E, _L), 0)

    for y in range(0, _YB, _G):
        xb = jnp.concatenate(
            [xb_all[:, y + j, :] for j in range(_G)], axis=1)
        etab = jnp.concatenate(
            [jnp.concatenate([eta_ref[y + j:y + j + 1, :]
                              for j in range(_G)], axis=1)] * _E,
            axis=0)

        logits = jax.lax.dot_general(
            Wt_bf, xb, (((0,), (0,)), ((), ())),
            preferred_element_type=jnp.float32)
        tadd = jnp.zeros((_E, _L), jnp.float32)
        for k in range(_E):
            tadd = jnp.where(etab == k, tcols[k], tadd)
        logits = logits + tadd

        mx = jnp.max(logits, axis=0, keepdims=True)
        mxb = jnp.concatenate([mx] * _E, axis=0)
        cand = jnp.where(logits == mxb, eidx_sub, _E)
        new_eta_row = jnp.min(cand, axis=0, keepdims=True)
        for j in range(_G):
            eta_out_ref[y + j:y + j + 1, :] = (
                new_eta_row[:, j * _NX:(j + 1) * _NX])

        netab = jnp.concatenate([new_eta_row] * _E, axis=0)
        mask = (netab == eidx_sub).astype(jnp.float32)
        mrep = jax.lax.dot_general(
            brep, mask.astype(jnp.bfloat16), (((1,), (0,)), ((), ())),
            preferred_element_type=jnp.float32
            ).astype(jnp.bfloat16)
        xrep = jnp.concatenate([xb] * _E, axis=0)
        xm = xrep * mrep

        badd = jax.lax.dot_general(
            b_ref[...], mask, (((1,), (0,)), ((), ())),
            preferred_element_type=jnp.float32)

        for p in range(_P):
            yv = jax.lax.dot_general(
                Wcat[p], xm, (((1,), (0,)), ((), ())),
                preferred_element_type=jnp.float32)
            res = yv + badd[p]
            for j in range(_G):
                out_ref[p, :, y + j, :] = res[:, j * _NX:(j + 1) * _NX]


def kernel(x, eta, W, b, Wt, Tmat):
    grid = (_NY // _YB,)
    out, new_eta = pl.pallas_call(
        _fused,
        grid=grid,
        in_specs=[
            pl.BlockSpec((_C, _YB, _NX), lambda i: (0, i, 0)),
            pl.BlockSpec((_YB, _NX), lambda i: (i, 0)),
            pl.BlockSpec((_P, _E, _C, _C), lambda i: (0, 0, 0, 0)),
            pl.BlockSpec((_P, _E, _C), lambda i: (0, 0, 0)),
            pl.BlockSpec((_C, _E), lambda i: (0, 0)),
            pl.BlockSpec((_E, _E), lambda i: (0, 0)),
        ],
        out_specs=[
            pl.BlockSpec((_P, _C, _YB, _NX), lambda i: (0, 0, i, 0)),
            pl.BlockSpec((_YB, _NX), lambda i: (i, 0)),
        ],
        out_shape=[
            jax.ShapeDtypeStruct((_P, _C, _NY, _NX), jnp.float32),
            jax.ShapeDtypeStruct((_NY, _NX), jnp.int32),
        ],
        compiler_params=pltpu.CompilerParams(
            dimension_semantics=("arbitrary",)),
    )(x, eta, W, b, Wt, Tmat)
    return out, new_eta


# speedup vs baseline: 3.3052x; 1.0344x over previous
import jax
import jax.numpy as jnp
from jax.experimental import pallas as pl
from jax.experimental.pallas import tpu as pltpu

_E = 8
_C = 128
_NY = 64
_NX = 128
_P = 2
_YB = 16
_G = 8
_L = _G * _NX


def _fused(x_ref, eta_ref, W_ref, b_ref, Wt_ref, Tmat_ref, out_ref, eta_out_ref):
    Wt_bf = Wt_ref[...].astype(jnp.bfloat16)
    xb_all = x_ref[...].astype(jnp.bfloat16)
    Wcat = [jnp.concatenate([W_ref[p, e].astype(jnp.bfloat16)
                             for e in range(_E)], axis=1)
            for p in range(_P)]

    brep = (jax.lax.broadcasted_iota(jnp.int32, (_E * _C, _E), 0) // _C ==
            jax.lax.broadcasted_iota(jnp.int32, (_E * _C, _E), 1)
            ).astype(jnp.bfloat16)
    tmat_t = Tmat_ref[...].T
    tcols = [jnp.broadcast_to(tmat_t[:, k:k + 1], (_E, _L)) for k in range(_E)]
    eidx_sub = jax.lax.broadcasted_iota(jnp.int32, (_E, _L), 0)

    for y in range(0, _YB, _G):
        xb = jnp.concatenate(
            [xb_all[:, y + j, :] for j in range(_G)], axis=1)
        etab = jnp.concatenate(
            [jnp.concatenate([eta_ref[y + j:y + j + 1, :]
                              for j in range(_G)], axis=1)] * _E,
            axis=0)

        logits = jax.lax.dot_general(
            Wt_bf, xb, (((0,), (0,)), ((), ())),
            preferred_element_type=jnp.float32)
        tadd = jnp.zeros((_E, _L), jnp.float32)
        for k in range(_E):
            tadd = jnp.where(etab == k, tcols[k], tadd)
        logits = logits + tadd

        mx = jnp.max(logits, axis=0, keepdims=True)
        mxb = jnp.concatenate([mx] * _E, axis=0)
        cand = jnp.where(logits == mxb, eidx_sub, _E)
        new_eta_row = jnp.min(cand, axis=0, keepdims=True)
        for j in range(_G):
            eta_out_ref[y + j:y + j + 1, :] = (
                new_eta_row[:, j * _NX:(j + 1) * _NX])

        netab = jnp.concatenate([new_eta_row] * _E, axis=0)
        mask = (netab == eidx_sub).astype(jnp.float32)
        mrep = jax.lax.dot_general(
            brep, mask.astype(jnp.bfloat16), (((1,), (0,)), ((), ())),
            preferred_element_type=jnp.float32
            ).astype(jnp.bfloat16)
        xrep = jnp.concatenate([xb] * _E, axis=0)
        xm = xrep * mrep

        badd = jax.lax.dot_general(
            b_ref[...], mask, (((1,), (0,)), ((), ())),
            preferred_element_type=jnp.float32)

        for p in range(_P):
            yv = jax.lax.dot_general(
                Wcat[p], xm, (((1,), (0,)), ((), ())),
                preferred_element_type=jnp.float32)
            res = yv + badd[p]
            for j in range(_G):
                out_ref[p, :, y + j, :] = res[:, j * _NX:(j + 1) * _NX]


def kernel(x, eta, W, b, Wt, Tmat):
    grid = (_NY // _YB,)
    out, new_eta = pl.pallas_call(
        _fused,
        grid=grid,
        in_specs=[
            pl.BlockSpec((_C, _YB, _NX), lambda i: (0, i, 0)),
            pl.BlockSpec((_YB, _NX), lambda i: (i, 0)),
            pl.BlockSpec((_P, _E, _C, _C), lambda i: (0, 0, 0, 0)),
            pl.BlockSpec((_P, _E, _C), lambda i: (0, 0, 0)),
            pl.BlockSpec((_C, _E), lambda i: (0, 0)),
            pl.BlockSpec((_E, _E), lambda i: (0, 0)),
        ],
        out_specs=[
            pl.BlockSpec((_P, _C, _YB, _NX), lambda i: (0, 0, i, 0)),
            pl.BlockSpec((_YB, _NX), lambda i: (i, 0)),
        ],
        out_shape=[
            jax.ShapeDtypeStruct((_P, _C, _NY, _NX), jnp.float32),
            jax.ShapeDtypeStruct((_NY, _NX), jnp.int32),
        ],
        compiler_params=pltpu.CompilerParams(
            dimension_semantics=("arbitrary",)),
    )(x, eta, W, b, Wt, Tmat)
    return out, new_eta
